# baseline (speedup 1.0000x reference)
import functools

import jax
import jax.numpy as jnp
from jax.experimental import pallas as pl
from jax.experimental.pallas import tpu as pltpu

_NFEAT = 128
_NHID = 64
_NCLASS = 32
_B = 2

_ROW_BLK = 256


def _feat_body(x_ref, w1_ref, wf_ref, bf_ref, h1_ref, init_ref):
    x = x_ref[...]
    h1_ref[...] = jnp.dot(x, w1_ref[...], preferred_element_type=jnp.float32)
    init_ref[...] = (
        jnp.dot(x, wf_ref[...], preferred_element_type=jnp.float32) + bf_ref[...]
    )


def _feat(x_flat, conv1_w, fc1_w, fc1_b):
    m = x_flat.shape[0]
    grid = pl.cdiv(m, _ROW_BLK)
    return pl.pallas_call(
        _feat_body,
        grid=(grid,),
        in_specs=[
            pl.BlockSpec((_ROW_BLK, _NFEAT), lambda i: (i, 0)),
            pl.BlockSpec((_NFEAT, _NHID), lambda i: (0, 0)),
            pl.BlockSpec((_NFEAT, _NCLASS), lambda i: (0, 0)),
            pl.BlockSpec((1, _NCLASS), lambda i: (0, 0)),
        ],
        out_specs=[
            pl.BlockSpec((_ROW_BLK, _NHID), lambda i: (i, 0)),
            pl.BlockSpec((_ROW_BLK, _NCLASS), lambda i: (i, 0)),
        ],
        out_shape=[
            jax.ShapeDtypeStruct((m, _NHID), jnp.float32),
            jax.ShapeDtypeStruct((m, _NCLASS), jnp.float32),
        ],
    )(x_flat, conv1_w, fc1_w, fc1_b.reshape(1, _NCLASS))


def _layer1_body(nmf_span, adj_ref, h1_ref, b1_ref, w2_ref, h2_ref):
    adj = adj_ref[...].astype(jnp.bfloat16)
    h1 = h1_ref[...].astype(jnp.bfloat16)
    t = jnp.dot(adj, h1, preferred_element_type=jnp.float32) + b1_ref[...]
    if nmf_span is None:
        s = jnp.maximum(t, 0.0)
    else:
        lo, hi = nmf_span
        col = jax.lax.broadcasted_iota(jnp.int32, t.shape, 1)
        keep_linear = (col >= lo) & (col < hi)
        s = jnp.where(keep_linear, t, jnp.maximum(t, 0.0))
    h2_ref[...] = jnp.dot(s, w2_ref[...], preferred_element_type=jnp.float32)


def _layer2_body(n_rows, adj_ref, h2_ref, init_ref, b2_ref, pool_ref):
    i = pl.program_id(0)
    adj = adj_ref[...].astype(jnp.bfloat16)
    h2 = h2_ref[...].astype(jnp.bfloat16)
    t = (
        jnp.dot(adj, h2, preferred_element_type=jnp.float32)
        + b2_ref[...]
        + init_ref[...]
    )
    rows = jax.lax.broadcasted_iota(jnp.int32, t.shape, 0) + i * _ROW_BLK
    t = jnp.where(rows < n_rows, t, -jnp.inf)
    m = jnp.max(t, axis=0, keepdims=True)
    m8 = jnp.broadcast_to(m, (8, t.shape[1]))

    @pl.when(i == 0)
    def _():
        pool_ref[...] = m8

    @pl.when(i > 0)
    def _():
        pool_ref[...] = jnp.maximum(pool_ref[...], m8)


def _gcn_pool(adj, h1, init, b1, b2, w2d, nmf_span):
    n = adj.shape[0]
    f1 = h1.shape[1]
    f2 = w2d.shape[1]
    grid = pl.cdiv(n, _ROW_BLK)
    params = pltpu.CompilerParams(vmem_limit_bytes=100 * 1024 * 1024)
    h2 = pl.pallas_call(
        functools.partial(_layer1_body, nmf_span),
        grid=(grid,),
        in_specs=[
            pl.BlockSpec((_ROW_BLK, n), lambda i: (i, 0)),
            pl.BlockSpec((n, f1), lambda i: (0, 0)),
            pl.BlockSpec((1, f1), lambda i: (0, 0)),
            pl.BlockSpec((f1, f2), lambda i: (0, 0)),
        ],
        out_specs=pl.BlockSpec((_ROW_BLK, f2), lambda i: (i, 0)),
        out_shape=jax.ShapeDtypeStruct((n, f2), jnp.float32),
        compiler_params=params,
    )(adj, h1, b1, w2d)
    pooled = pl.pallas_call(
        functools.partial(_layer2_body, n),
        grid=(grid,),
        in_specs=[
            pl.BlockSpec((_ROW_BLK, n), lambda i: (i, 0)),
            pl.BlockSpec((n, f2), lambda i: (0, 0)),
            pl.BlockSpec((_ROW_BLK, f2), lambda i: (i, 0)),
            pl.BlockSpec((1, f2), lambda i: (0, 0)),
        ],
        out_specs=pl.BlockSpec((8, f2), lambda i: (0, 0)),
        out_shape=jax.ShapeDtypeStruct((8, f2), jnp.float32),
        compiler_params=params,
    )(adj, h2, init, b2)
    return pooled[0]


def _head_body(x_ref, w2_ref, b2_ref, w3_ref, b3_ref, w4_ref, b4_ref,
               w5_ref, b5_ref, out_ref):
    x = x_ref[...]
    x = jnp.maximum(jnp.dot(x, w2_ref[...], preferred_element_type=jnp.float32)
                    + b2_ref[...], 0.0)
    x = jnp.maximum(jnp.dot(x, w3_ref[...], preferred_element_type=jnp.float32)
                    + b3_ref[...], 0.0)
    x = jnp.maximum(jnp.dot(x, w4_ref[...], preferred_element_type=jnp.float32)
                    + b4_ref[...], 0.0)
    d = jnp.dot(x, w5_ref[...], preferred_element_type=jnp.float32) + b5_ref[...]
    out_ref[...] = d + jnp.zeros((8, 8), jnp.float32)


def _head(x8, fc2_w, fc2_b, fc3_w, fc3_b, fc4_w, fc4_b, fc5_w, fc5_b):
    full = lambda shape: pl.BlockSpec(shape, lambda: (0,) * len(shape))
    return pl.pallas_call(
        _head_body,
        in_specs=[
            full((8, 2 * _NCLASS)),
            full(fc2_w.shape), full((1, _NCLASS)),
            full(fc3_w.shape), full((1, 64)),
            full(fc4_w.shape), full((1, 32)),
            full(fc5_w.shape), full((1, 1)),
        ],
        out_specs=full((8, 8)),
        out_shape=jax.ShapeDtypeStruct((8, 8), jnp.float32),
    )(x8, fc2_w, fc2_b.reshape(1, -1), fc3_w, fc3_b.reshape(1, -1),
      fc4_w, fc4_b.reshape(1, -1), fc5_w, fc5_b.reshape(1, -1))


def _colstack(flat, n_sets, n_nodes, width):
    a = flat.reshape(n_sets * _B, n_nodes, width)
    return a.transpose(1, 0, 2).reshape(n_nodes, n_sets * _B * width)


def kernel(solute_ACE, solvent_ACE, solute_adj, solvent_adj_ACE, solute_NMF,
           solvent_NMF, solvent_adj_NMF, solute_wat, solvent_wat,
           solvent_adj_wat, fc1_w, fc1_b, conv1_w, conv1_b, conv2_w, conv2_b,
           fc2_w, fc2_b, fc3_w, fc3_b, fc4_w, fc4_b, fc5_w, fc5_b):
    n_su = solute_ACE.shape[1]

    su_x = jnp.concatenate(
        [solute_ACE, solute_NMF, solute_wat], axis=0).reshape(-1, _NFEAT)
    su_h1f, su_initf = _feat(su_x, conv1_w, fc1_w, fc1_b)
    ace_h1f, ace_initf = _feat(solvent_ACE.reshape(-1, _NFEAT), conv1_w, fc1_w, fc1_b)
    nmf_h1f, nmf_initf = _feat(solvent_NMF.reshape(-1, _NFEAT), conv1_w, fc1_w, fc1_b)
    wat_h1f, wat_initf = _feat(solvent_wat.reshape(-1, _NFEAT), conv1_w, fc1_w, fc1_b)

    su_h1 = _colstack(su_h1f, 3, n_su, _NHID)
    su_init = _colstack(su_initf, 3, n_su, _NCLASS)
    sv_h1 = {
        'ACE': _colstack(ace_h1f, 1, solvent_ACE.shape[1], _NHID),
        'NMF': _colstack(nmf_h1f, 1, solvent_NMF.shape[1], _NHID),
        'wat': _colstack(wat_h1f, 1, solvent_wat.shape[1], _NHID),
    }
    sv_init = {
        'ACE': _colstack(ace_initf, 1, solvent_ACE.shape[1], _NCLASS),
        'NMF': _colstack(nmf_initf, 1, solvent_NMF.shape[1], _NCLASS),
        'wat': _colstack(wat_initf, 1, solvent_wat.shape[1], _NCLASS),
    }

    b1_sv = jnp.tile(conv1_b, _B).reshape(1, -1)
    b2_sv = jnp.tile(conv2_b, _B).reshape(1, -1)
    b1_su = jnp.tile(conv1_b, 3 * _B).reshape(1, -1)
    b2_su = jnp.tile(conv2_b, 3 * _B).reshape(1, -1)
    w2_sv = jnp.kron(jnp.eye(_B, dtype=jnp.float32), conv2_w)
    w2_su = jnp.kron(jnp.eye(3 * _B, dtype=jnp.float32), conv2_w)

    p_su = _gcn_pool(solute_adj, su_h1, su_init, b1_su, b2_su, w2_su,
                     nmf_span=(_B * _NHID, 2 * _B * _NHID))
    p_ace = _gcn_pool(solvent_adj_ACE, sv_h1['ACE'], sv_init['ACE'],
                      b1_sv, b2_sv, w2_sv, nmf_span=None)
    p_nmf = _gcn_pool(solvent_adj_NMF, sv_h1['NMF'], sv_init['NMF'],
                      b1_sv, b2_sv, w2_sv, nmf_span=None)
    p_wat = _gcn_pool(solvent_adj_wat, sv_h1['wat'], sv_init['wat'],
                      b1_sv, b2_sv, w2_sv, nmf_span=None)

    c = _NCLASS
    rows = [
        jnp.concatenate([p_su[0 * c:1 * c], p_ace[0:c]]),
        jnp.concatenate([p_su[1 * c:2 * c], p_ace[c:2 * c]]),
        jnp.concatenate([p_su[2 * c:3 * c], p_nmf[0:c]]),
        jnp.concatenate([p_su[3 * c:4 * c], p_nmf[c:2 * c]]),
        jnp.concatenate([p_su[4 * c:5 * c], p_wat[0:c]]),
        jnp.concatenate([p_su[5 * c:6 * c], p_wat[c:2 * c]]),
    ]
    x8 = jnp.pad(jnp.stack(rows), ((0, 2), (0, 0)))
    out = _head(x8, fc2_w, fc2_b, fc3_w, fc3_b, fc4_w, fc4_b, fc5_w, fc5_b)
    return out[:6, :1]


# speedup vs baseline: 1.2097x; 1.2097x over previous
import functools

import jax
import jax.numpy as jnp
from jax.experimental import pallas as pl
from jax.experimental.pallas import tpu as pltpu

_NFEAT = 128
_NHID = 64
_NCLASS = 32
_B = 2

_ROW_BLK = 256


def _feat_body(n_sets, w1_ref, wf_ref, bf_ref, *refs):
    x_refs = refs[:n_sets]
    h1_ref, init_ref = refs[n_sets], refs[n_sets + 1]
    w1 = w1_ref[...]
    wf = wf_ref[...]
    bf = bf_ref[...]
    h1_parts = []
    init_parts = []
    for x_ref in x_refs:
        for b in range(_B):
            xb = x_ref[b]
            h1_parts.append(
                jnp.dot(xb, w1, preferred_element_type=jnp.float32))
            init_parts.append(
                jnp.dot(xb, wf, preferred_element_type=jnp.float32) + bf)
    h1_ref[...] = jnp.concatenate(h1_parts, axis=1)
    init_ref[...] = jnp.concatenate(init_parts, axis=1)


def _feat(xs, conv1_w, fc1_w, fc1_b):
    n_sets = len(xs)
    n = xs[0].shape[1]
    grid = pl.cdiv(n, _ROW_BLK)
    return pl.pallas_call(
        functools.partial(_feat_body, n_sets),
        grid=(grid,),
        in_specs=[
            pl.BlockSpec((_NFEAT, _NHID), lambda i: (0, 0)),
            pl.BlockSpec((_NFEAT, _NCLASS), lambda i: (0, 0)),
            pl.BlockSpec((1, _NCLASS), lambda i: (0, 0)),
        ] + [
            pl.BlockSpec((_B, _ROW_BLK, _NFEAT), lambda i: (0, i, 0))
            for _ in range(n_sets)
        ],
        out_specs=[
            pl.BlockSpec((_ROW_BLK, n_sets * _B * _NHID), lambda i: (i, 0)),
            pl.BlockSpec((_ROW_BLK, n_sets * _B * _NCLASS), lambda i: (i, 0)),
        ],
        out_shape=[
            jax.ShapeDtypeStruct((n, n_sets * _B * _NHID), jnp.float32),
            jax.ShapeDtypeStruct((n, n_sets * _B * _NCLASS), jnp.float32),
        ],
    )(conv1_w, fc1_w, fc1_b.reshape(1, _NCLASS), *xs)


def _layer1_body(nmf_span, adj_ref, h1_ref, b1_ref, w2_ref, h2_ref):
    t = jnp.dot(adj_ref[...], h1_ref[...],
                preferred_element_type=jnp.float32) + b1_ref[...]
    if nmf_span is None:
        s = jnp.maximum(t, 0.0)
    else:
        lo, hi = nmf_span
        col = jax.lax.broadcasted_iota(jnp.int32, t.shape, 1)
        keep_linear = (col >= lo) & (col < hi)
        s = jnp.where(keep_linear, t, jnp.maximum(t, 0.0))
    h2_ref[...] = jnp.dot(s, w2_ref[...], preferred_element_type=jnp.float32)


def _layer2_body(n_rows, adj_ref, h2_ref, init_ref, b2_ref, pool_ref):
    i = pl.program_id(0)
    t = (
        jnp.dot(adj_ref[...], h2_ref[...], preferred_element_type=jnp.float32)
        + b2_ref[...]
        + init_ref[...]
    )
    rows = jax.lax.broadcasted_iota(jnp.int32, t.shape, 0) + i * _ROW_BLK
    t = jnp.where(rows < n_rows, t, -jnp.inf)
    m = jnp.max(t, axis=0, keepdims=True)
    m8 = jnp.broadcast_to(m, (8, t.shape[1]))

    @pl.when(i == 0)
    def _():
        pool_ref[...] = m8

    @pl.when(i > 0)
    def _():
        pool_ref[...] = jnp.maximum(pool_ref[...], m8)


def _gcn_pool(adj, h1, init, b1, b2, w2d, nmf_span):
    n = adj.shape[0]
    f1 = h1.shape[1]
    f2 = w2d.shape[1]
    grid = pl.cdiv(n, _ROW_BLK)
    params = pltpu.CompilerParams(vmem_limit_bytes=100 * 1024 * 1024)
    h2 = pl.pallas_call(
        functools.partial(_layer1_body, nmf_span),
        grid=(grid,),
        in_specs=[
            pl.BlockSpec((_ROW_BLK, n), lambda i: (i, 0)),
            pl.BlockSpec((n, f1), lambda i: (0, 0)),
            pl.BlockSpec((1, f1), lambda i: (0, 0)),
            pl.BlockSpec((f1, f2), lambda i: (0, 0)),
        ],
        out_specs=pl.BlockSpec((_ROW_BLK, f2), lambda i: (i, 0)),
        out_shape=jax.ShapeDtypeStruct((n, f2), jnp.float32),
        compiler_params=params,
    )(adj, h1, b1, w2d)
    pooled = pl.pallas_call(
        functools.partial(_layer2_body, n),
        grid=(grid,),
        in_specs=[
            pl.BlockSpec((_ROW_BLK, n), lambda i: (i, 0)),
            pl.BlockSpec((n, f2), lambda i: (0, 0)),
            pl.BlockSpec((_ROW_BLK, f2), lambda i: (i, 0)),
            pl.BlockSpec((1, f2), lambda i: (0, 0)),
        ],
        out_specs=pl.BlockSpec((8, f2), lambda i: (0, 0)),
        out_shape=jax.ShapeDtypeStruct((8, f2), jnp.float32),
        compiler_params=params,
    )(adj, h2, init, b2)
    return pooled[0]


def _head_body(x_ref, w2_ref, b2_ref, w3_ref, b3_ref, w4_ref, b4_ref,
               w5_ref, b5_ref, out_ref):
    x = x_ref[...]
    x = jnp.maximum(jnp.dot(x, w2_ref[...], preferred_element_type=jnp.float32)
                    + b2_ref[...], 0.0)
    x = jnp.maximum(jnp.dot(x, w3_ref[...], preferred_element_type=jnp.float32)
                    + b3_ref[...], 0.0)
    x = jnp.maximum(jnp.dot(x, w4_ref[...], preferred_element_type=jnp.float32)
                    + b4_ref[...], 0.0)
    d = jnp.dot(x, w5_ref[...], preferred_element_type=jnp.float32) + b5_ref[...]
    out_ref[...] = d + jnp.zeros((8, 8), jnp.float32)


def _head(x8, fc2_w, fc2_b, fc3_w, fc3_b, fc4_w, fc4_b, fc5_w, fc5_b):
    full = lambda shape: pl.BlockSpec(shape, lambda: (0,) * len(shape))
    return pl.pallas_call(
        _head_body,
        in_specs=[
            full((8, 2 * _NCLASS)),
            full(fc2_w.shape), full((1, _NCLASS)),
            full(fc3_w.shape), full((1, 64)),
            full(fc4_w.shape), full((1, 32)),
            full(fc5_w.shape), full((1, 1)),
        ],
        out_specs=full((8, 8)),
        out_shape=jax.ShapeDtypeStruct((8, 8), jnp.float32),
    )(x8, fc2_w, fc2_b.reshape(1, -1), fc3_w, fc3_b.reshape(1, -1),
      fc4_w, fc4_b.reshape(1, -1), fc5_w, fc5_b.reshape(1, -1))


def kernel(solute_ACE, solvent_ACE, solute_adj, solvent_adj_ACE, solute_NMF,
           solvent_NMF, solvent_adj_NMF, solute_wat, solvent_wat,
           solvent_adj_wat, fc1_w, fc1_b, conv1_w, conv1_b, conv2_w, conv2_b,
           fc2_w, fc2_b, fc3_w, fc3_b, fc4_w, fc4_b, fc5_w, fc5_b):
    su_h1, su_init = _feat([solute_ACE, solute_NMF, solute_wat],
                           conv1_w, fc1_w, fc1_b)
    sv_h1, sv_init = {}, {}
    sv_h1['ACE'], sv_init['ACE'] = _feat([solvent_ACE], conv1_w, fc1_w, fc1_b)
    sv_h1['NMF'], sv_init['NMF'] = _feat([solvent_NMF], conv1_w, fc1_w, fc1_b)
    sv_h1['wat'], sv_init['wat'] = _feat([solvent_wat], conv1_w, fc1_w, fc1_b)

    b1_sv = jnp.tile(conv1_b, _B).reshape(1, -1)
    b2_sv = jnp.tile(conv2_b, _B).reshape(1, -1)
    b1_su = jnp.tile(conv1_b, 3 * _B).reshape(1, -1)
    b2_su = jnp.tile(conv2_b, 3 * _B).reshape(1, -1)
    w2_sv = jnp.kron(jnp.eye(_B, dtype=jnp.float32), conv2_w)
    w2_su = jnp.kron(jnp.eye(3 * _B, dtype=jnp.float32), conv2_w)

    p_su = _gcn_pool(solute_adj, su_h1, su_init, b1_su, b2_su, w2_su,
                     nmf_span=(_B * _NHID, 2 * _B * _NHID))
    p_ace = _gcn_pool(solvent_adj_ACE, sv_h1['ACE'], sv_init['ACE'],
                      b1_sv, b2_sv, w2_sv, nmf_span=None)
    p_nmf = _gcn_pool(solvent_adj_NMF, sv_h1['NMF'], sv_init['NMF'],
                      b1_sv, b2_sv, w2_sv, nmf_span=None)
    p_wat = _gcn_pool(solvent_adj_wat, sv_h1['wat'], sv_init['wat'],
                      b1_sv, b2_sv, w2_sv, nmf_span=None)

    c = _NCLASS
    rows = [
        jnp.concatenate([p_su[0 * c:1 * c], p_ace[0:c]]),
        jnp.concatenate([p_su[1 * c:2 * c], p_ace[c:2 * c]]),
        jnp.concatenate([p_su[2 * c:3 * c], p_nmf[0:c]]),
        jnp.concatenate([p_su[3 * c:4 * c], p_nmf[c:2 * c]]),
        jnp.concatenate([p_su[4 * c:5 * c], p_wat[0:c]]),
        jnp.concatenate([p_su[5 * c:6 * c], p_wat[c:2 * c]]),
    ]
    x8 = jnp.pad(jnp.stack(rows), ((0, 2), (0, 0)))
    out = _head(x8, fc2_w, fc2_b, fc3_w, fc3_b, fc4_w, fc4_b, fc5_w, fc5_b)
    return out[:6, :1]
